# baseline (speedup 1.0000x reference)
import math

import jax
import jax.numpy as jnp
from jax import lax
from jax.experimental import pallas as pl
from jax.experimental.pallas import tpu as pltpu

_SQRT_2_OVER_PI = math.sqrt(2.0 / math.pi)

_BM = 128
_NBUF = 8


def _gelu(x):
    inner = _SQRT_2_OVER_PI * (x + 0.044715 * (x * x * x))
    return 0.5 * x * (1.0 + jnp.tanh(inner))


def _pipe_body(x_hbm, o_hbm, ibuf, obuf, isem, osem):
    n = x_hbm.shape[0] // _BM

    for s in range(_NBUF):
        pltpu.make_async_copy(
            x_hbm.at[pl.ds(s * _BM, _BM), :], ibuf.at[s], isem.at[s]
        ).start()

    def step(i, carry):
        s = lax.rem(i, _NBUF)
        pltpu.make_async_copy(
            x_hbm.at[pl.ds(i * _BM, _BM), :], ibuf.at[s], isem.at[s]
        ).wait()

        @pl.when(i >= _NBUF)
        def _():
            pltpu.make_async_copy(
                obuf.at[s], o_hbm.at[pl.ds((i - _NBUF) * _BM, _BM), :],
                osem.at[s],
            ).wait()

        obuf[s] = _gelu(ibuf[s])
        pltpu.make_async_copy(
            obuf.at[s], o_hbm.at[pl.ds(i * _BM, _BM), :], osem.at[s]
        ).start()

        @pl.when(i + _NBUF < n)
        def _():
            pltpu.make_async_copy(
                x_hbm.at[pl.ds((i + _NBUF) * _BM, _BM), :], ibuf.at[s],
                isem.at[s],
            ).start()

        return carry

    lax.fori_loop(0, n, step, 0)

    for k in range(_NBUF):
        i = n - _NBUF + k
        s = i % _NBUF
        pltpu.make_async_copy(
            obuf.at[s], o_hbm.at[pl.ds(i * _BM, _BM), :], osem.at[s]
        ).wait()


def kernel(x, logit_decay, log_tau, log_blend):
    del logit_decay, log_tau, log_blend
    B, T, D = x.shape
    x2 = x.reshape(B * T, D)
    out = pl.pallas_call(
        _pipe_body,
        in_specs=[pl.BlockSpec(memory_space=pl.ANY)],
        out_specs=pl.BlockSpec(memory_space=pl.ANY),
        out_shape=jax.ShapeDtypeStruct(x2.shape, x2.dtype),
        scratch_shapes=[
            pltpu.VMEM((_NBUF, _BM, D), jnp.float32),
            pltpu.VMEM((_NBUF, _BM, D), jnp.float32),
            pltpu.SemaphoreType.DMA((_NBUF,)),
            pltpu.SemaphoreType.DMA((_NBUF,)),
        ],
    )(x2)
    return out.reshape(B, T, D)


# speedup vs baseline: 1.0029x; 1.0029x over previous
import math

import jax
import jax.numpy as jnp
from jax import lax
from jax.experimental import pallas as pl
from jax.experimental.pallas import tpu as pltpu

_SQRT_2_OVER_PI = math.sqrt(2.0 / math.pi)

_BM = 512
_NBUF = 4


def _gelu(x):
    inner = _SQRT_2_OVER_PI * (x + 0.044715 * (x * x * x))
    return 0.5 * x * (1.0 + jnp.tanh(inner))


def _pipe_body(x_hbm, o_hbm, ibuf, obuf, isem, osem):
    n = x_hbm.shape[0] // _BM

    for s in range(_NBUF):
        pltpu.make_async_copy(
            x_hbm.at[pl.ds(s * _BM, _BM), :], ibuf.at[s], isem.at[s]
        ).start()

    def step(i, carry):
        s = lax.rem(i, _NBUF)
        pltpu.make_async_copy(
            x_hbm.at[pl.ds(i * _BM, _BM), :], ibuf.at[s], isem.at[s]
        ).wait()

        @pl.when(i >= _NBUF)
        def _():
            pltpu.make_async_copy(
                obuf.at[s], o_hbm.at[pl.ds((i - _NBUF) * _BM, _BM), :],
                osem.at[s],
            ).wait()

        obuf[s] = _gelu(ibuf[s])
        pltpu.make_async_copy(
            obuf.at[s], o_hbm.at[pl.ds(i * _BM, _BM), :], osem.at[s]
        ).start()

        @pl.when(i + _NBUF < n)
        def _():
            pltpu.make_async_copy(
                x_hbm.at[pl.ds((i + _NBUF) * _BM, _BM), :], ibuf.at[s],
                isem.at[s],
            ).start()

        return carry

    lax.fori_loop(0, n, step, 0)

    for k in range(_NBUF):
        i = n - _NBUF + k
        s = i % _NBUF
        pltpu.make_async_copy(
            obuf.at[s], o_hbm.at[pl.ds(i * _BM, _BM), :], osem.at[s]
        ).wait()


def kernel(x, logit_decay, log_tau, log_blend):
    del logit_decay, log_tau, log_blend
    B, T, D = x.shape
    x2 = x.reshape(B * T, D)
    out = pl.pallas_call(
        _pipe_body,
        in_specs=[pl.BlockSpec(memory_space=pl.ANY)],
        out_specs=pl.BlockSpec(memory_space=pl.ANY),
        out_shape=jax.ShapeDtypeStruct(x2.shape, x2.dtype),
        scratch_shapes=[
            pltpu.VMEM((_NBUF, _BM, D), jnp.float32),
            pltpu.VMEM((_NBUF, _BM, D), jnp.float32),
            pltpu.SemaphoreType.DMA((_NBUF,)),
            pltpu.SemaphoreType.DMA((_NBUF,)),
        ],
    )(x2)
    return out.reshape(B, T, D)
